# baseline (speedup 1.0000x reference)
import functools

import jax
import jax.numpy as jnp
from jax import lax
from jax.experimental import pallas as pl
from jax.experimental.pallas import tpu as pltpu
from jax.experimental.pallas import tpu_sc as plsc

N = 10000
E = 320000
DIM = 16
EPS = 1e-5

NC = 2
NS = 16
NW = NC * NS
EPW = E // NW
IDXW = 125
ROWS_PER_W = EPW // IDXW
CHUNK = 1000
DMAS_PER_CHUNK = CHUNK // IDXW
CHUNKS_PER_W = EPW // CHUNK

f32 = jnp.float32


def relu(v):
    return jnp.maximum(v, 0.0)


def _make_gather(n_idx):
    mesh = plsc.VectorSubcoreMesh(core_axis_name="c", subcore_axis_name="s", num_cores=NC, num_subcores=NS)
    out_type = [jax.ShapeDtypeStruct((E, DIM), f32)] * n_idx
    scratch = [pltpu.VMEM((ROWS_PER_W, IDXW), jnp.int32)] * n_idx
    scratch += [pltpu.VMEM((CHUNK, DIM), f32), pltpu.SemaphoreType.DMA]

    @functools.partial(pl.kernel, out_type=out_type, mesh=mesh,
                       scratch_types=scratch,
                       compiler_params=pltpu.CompilerParams(
                           use_tc_tiling_on_sc=False))
    def gather(*refs):
        table = refs[0]
        idx_hbm = refs[1:1 + n_idx]
        outs = refs[1 + n_idx:1 + 2 * n_idx]
        idx_v = refs[1 + 2 * n_idx:1 + 3 * n_idx]
        rows_v = refs[1 + 3 * n_idx]
        sem = refs[2 + 3 * n_idx]

        wid = lax.axis_index("s") * NC + lax.axis_index("c")
        irow0 = wid * ROWS_PER_W
        base = wid * EPW
        for t in range(n_idx):
            pltpu.sync_copy(idx_hbm[t].at[pl.ds(irow0, ROWS_PER_W), :],
                            idx_v[t])

        def body(j, carry):
            for t in range(n_idx):
                descs = []
                for k in range(DMAS_PER_CHUNK):
                    irow = j * DMAS_PER_CHUNK + k
                    d = pltpu.async_copy(
                        table.at[idx_v[t].at[irow]],
                        rows_v.at[pl.ds(k * IDXW, IDXW), :], sem)
                    descs.append(d)
                for d in descs:
                    d.wait()
                pltpu.sync_copy(rows_v,
                                outs[t].at[pl.ds(base + j * CHUNK, CHUNK), :])
            return carry

        lax.fori_loop(0, CHUNKS_PER_W, body, 0)

    return gather


_gather1 = _make_gather(1)
_gather2 = _make_gather(2)


def _scatter_add_kernel():
    mesh = plsc.VectorSubcoreMesh(core_axis_name="c", subcore_axis_name="s", num_cores=NC, num_subcores=NS)
    npc = N // NS

    @functools.partial(
        pl.kernel,
        out_type=jax.ShapeDtypeStruct((NC, N, DIM), f32),
        mesh=mesh,
        compiler_params=pltpu.CompilerParams(use_tc_tiling_on_sc=False),
        scratch_types=[
            pltpu.VMEM_SHARED((N, DIM), f32),
            pltpu.VMEM((ROWS_PER_W, IDXW), jnp.int32),
            pltpu.VMEM((CHUNK, DIM), f32),
        ])
    def scatter_add(msg_hbm, dst_hbm, zeros_hbm, out_hbm, acc_sh, idx_v,
                    msg_v):
        cid = lax.axis_index("c")
        sid = lax.axis_index("s")
        wid = sid * NC + cid
        pltpu.sync_copy(zeros_hbm.at[pl.ds(sid * npc, npc), :],
                        acc_sh.at[pl.ds(sid * npc, npc), :])
        plsc.subcore_barrier()

        pltpu.sync_copy(dst_hbm.at[pl.ds(wid * ROWS_PER_W, ROWS_PER_W), :],
                        idx_v)
        base = wid * EPW

        def body(j, carry):
            pltpu.sync_copy(msg_hbm.at[pl.ds(base + j * CHUNK, CHUNK), :],
                            msg_v)
            for k in range(DMAS_PER_CHUNK):
                irow = j * DMAS_PER_CHUNK + k
                pltpu.sync_copy(msg_v.at[pl.ds(k * IDXW, IDXW), :],
                                acc_sh.at[idx_v.at[irow]], add=True)
            return carry

        lax.fori_loop(0, CHUNKS_PER_W, body, 0)
        plsc.subcore_barrier()
        pltpu.sync_copy(acc_sh.at[pl.ds(sid * npc, npc), :],
                        out_hbm.at[cid, pl.ds(sid * npc, npc), :])

    return scatter_add


_scatter_add = _scatter_add_kernel()


BLK = 512
NBLK = E // BLK


def _mean_rstd(stats, d):
    s = stats[0:1, :d]
    sq = stats[1:2, :d]
    mean = s * (1.0 / N)
    var = sq * (1.0 / N) - mean * mean
    return mean, lax.rsqrt(var + EPS)


def _vpu_matmul(x, wt):
    k = wt.shape[0]
    acc = x[:, 0:1] * wt[0:1, :]
    for i in range(1, k):
        acc = acc + x[:, i:i + 1] * wt[i:i + 1, :]
    return acc


def _conv_body(ci, co, stats_ref, ea_ref, xr_ref, w1t, b1, w2t, b2, w3t, b3,
               msg_ref):
    ea = ea_ref[...]
    h1 = relu(_vpu_matmul(ea, w1t[...]) + b1[...])
    h2 = relu(_vpu_matmul(h1, w2t[...]) + b2[...])
    w = relu(jnp.dot(h2, w3t[...], preferred_element_type=f32,
                     precision=lax.Precision.HIGHEST) + b3[...])
    mean, rstd = _mean_rstd(stats_ref[...], ci)
    if ci == 1:
        xn = (xr_ref[:, 0:1] - mean) * rstd
        msg_ref[...] = xn * w
    else:
        xn = (xr_ref[...] - mean) * rstd
        if co == DIM:
            acc = xn[:, 0:1] * w[:, 0:DIM]
            for i in range(1, DIM):
                acc = acc + xn[:, i:i + 1] * w[:, i * DIM:(i + 1) * DIM]
            msg_ref[...] = acc
        else:
            s = jnp.sum(xn * w, axis=1, keepdims=True)
            msg_ref[...] = jnp.concatenate(
                [s, jnp.zeros((s.shape[0], DIM - 1), f32)], axis=1)


def _make_conv(ci, co):
    wdim = ci * co
    body = functools.partial(_conv_body, ci, co)
    return pl.pallas_call(
        body,
        grid=(NBLK,),
        in_specs=[
            pl.BlockSpec((2, DIM), lambda e: (0, 0)),
            pl.BlockSpec((BLK, 2), lambda e: (e, 0)),
            pl.BlockSpec((BLK, DIM), lambda e: (e, 0)),
            pl.BlockSpec((2, 4), lambda e: (0, 0)),
            pl.BlockSpec((1, 4), lambda e: (0, 0)),
            pl.BlockSpec((4, DIM), lambda e: (0, 0)),
            pl.BlockSpec((1, DIM), lambda e: (0, 0)),
            pl.BlockSpec((DIM, wdim), lambda e: (0, 0)),
            pl.BlockSpec((1, wdim), lambda e: (0, 0)),
        ],
        out_specs=pl.BlockSpec((BLK, DIM), lambda e: (e, 0)),
        out_shape=jax.ShapeDtypeStruct((E, DIM), f32),
    )


_conv_first = _make_conv(1, DIM)
_conv_mid = _make_conv(DIM, DIM)
_conv_last = _make_conv(DIM, 1)


def _em_body(xdim, co, residual, ea_ref, xr_ref, xc_ref, w1at, w1bt, w1ct,
             b1, gamma, beta, w2t, b2, out_ref):
    ea = ea_ref[...]
    if xdim == 1:
        acc = xr_ref[:, 0:1] * w1at[0:1, :] + xc_ref[:, 0:1] * w1bt[0:1, :]
    else:
        acc = _vpu_matmul(xr_ref[...], w1at[...]) + _vpu_matmul(
            xc_ref[...], w1bt[...])
    acc = acc + _vpu_matmul(ea, w1ct[...]) + b1[...]
    h = relu(acc)
    m = jnp.mean(h, axis=1, keepdims=True)
    v = jnp.mean((h - m) * (h - m), axis=1, keepdims=True)
    hn = (h - m) * lax.rsqrt(v + EPS) * gamma[...] + beta[...]
    o = relu(_vpu_matmul(hn, w2t[...]) + b2[...])
    if residual:
        out_ref[...] = o + ea
    else:
        out_ref[...] = jnp.concatenate(
            [o, jnp.zeros((o.shape[0], DIM - co), f32)], axis=1)


def _make_em(xdim, co, residual):
    body = functools.partial(_em_body, xdim, co, residual)
    odim = 2 if residual else DIM
    return pl.pallas_call(
        body,
        grid=(NBLK,),
        in_specs=[
            pl.BlockSpec((BLK, 2), lambda e: (e, 0)),
            pl.BlockSpec((BLK, DIM), lambda e: (e, 0)),
            pl.BlockSpec((BLK, DIM), lambda e: (e, 0)),
            pl.BlockSpec((xdim, DIM), lambda e: (0, 0)),
            pl.BlockSpec((xdim, DIM), lambda e: (0, 0)),
            pl.BlockSpec((2, DIM), lambda e: (0, 0)),
            pl.BlockSpec((1, DIM), lambda e: (0, 0)),
            pl.BlockSpec((1, DIM), lambda e: (0, 0)),
            pl.BlockSpec((1, DIM), lambda e: (0, 0)),
            pl.BlockSpec((DIM, co), lambda e: (0, 0)),
            pl.BlockSpec((1, co), lambda e: (0, 0)),
        ],
        out_specs=pl.BlockSpec((BLK, odim), lambda e: (e, 0)),
        out_shape=jax.ShapeDtypeStruct((E, odim), f32),
    )


_em_mid = _make_em(DIM, 2, True)
_em_last = _make_em(1, 1, False)


def _stats0_body(x_ref, out_ref):
    x = x_ref[...]
    s = jnp.sum(x)
    sq = jnp.sum(x * x)
    out_ref[...] = jnp.concatenate(
        [jnp.full((1, DIM), s, f32), jnp.full((1, DIM), sq, f32)], axis=0)


_stats0 = pl.pallas_call(
    _stats0_body,
    out_shape=jax.ShapeDtypeStruct((2, DIM), f32),
)


NODE_BLK = 1000
NODE_NBLK = N // NODE_BLK


def _node_body(ci, p_ref, x_ref, stats_ref, roott, bias, x_out, st_out):
    agg = p_ref[0] + p_ref[1]
    mean, rstd = _mean_rstd(stats_ref[...], ci)
    if ci == 1:
        xn = (x_ref[...] - mean) * rstd
        r = xn * roott[...]
        x_new = relu(agg + r + bias[...]) + x_ref[...]
    else:
        xn = (x_ref[...] - mean) * rstd
        r = _vpu_matmul(xn, roott[...])
        x_new = relu(agg + r + bias[...]) + x_ref[...]
    x_out[...] = x_new
    @pl.when(pl.program_id(0) == 0)
    def _():
        st_out[...] = jnp.zeros((2, DIM), f32)
    s = jnp.sum(x_new, axis=0, keepdims=True)
    sq = jnp.sum(x_new * x_new, axis=0, keepdims=True)
    st_out[...] += jnp.concatenate([s, sq], axis=0)


def _make_node(ci):
    xdim = 1 if ci == 1 else DIM
    return pl.pallas_call(
        functools.partial(_node_body, ci),
        grid=(NODE_NBLK,),
        in_specs=[
            pl.BlockSpec((NC, NODE_BLK, DIM), lambda n: (0, n, 0)),
            pl.BlockSpec((NODE_BLK, xdim), lambda n: (n, 0)),
            pl.BlockSpec((2, DIM), lambda n: (0, 0)),
            pl.BlockSpec((ci, DIM), lambda n: (0, 0)),
            pl.BlockSpec((1, DIM), lambda n: (0, 0)),
        ],
        out_specs=[
            pl.BlockSpec((NODE_BLK, DIM), lambda n: (n, 0)),
            pl.BlockSpec((2, DIM), lambda n: (0, 0)),
        ],
        out_shape=[
            jax.ShapeDtypeStruct((N, DIM), f32),
            jax.ShapeDtypeStruct((2, DIM), f32),
        ],
    )


_node_mid = _make_node(DIM)
_node_first = _make_node(1)


def _node6_body(p_ref, x_ref, stats_ref, roott, bias, x_out):
    agg = p_ref[0, :, 0:1] + p_ref[1, :, 0:1]
    mean, rstd = _mean_rstd(stats_ref[...], DIM)
    xn = (x_ref[...] - mean) * rstd
    r = jnp.sum(xn * roott[...], axis=1, keepdims=True)
    x6 = relu(agg + r + bias[:, 0:1])
    x_out[...] = jnp.concatenate(
        [x6, jnp.zeros((NODE_BLK, DIM - 1), f32)], axis=1)


_node_last = pl.pallas_call(
    _node6_body,
    grid=(NODE_NBLK,),
    in_specs=[
        pl.BlockSpec((NC, NODE_BLK, DIM), lambda n: (0, n, 0)),
        pl.BlockSpec((NODE_BLK, DIM), lambda n: (n, 0)),
        pl.BlockSpec((2, DIM), lambda n: (0, 0)),
        pl.BlockSpec((1, DIM), lambda n: (0, 0)),
        pl.BlockSpec((1, DIM), lambda n: (0, 0)),
    ],
    out_specs=pl.BlockSpec((NODE_BLK, DIM), lambda n: (n, 0)),
    out_shape=jax.ShapeDtypeStruct((N, DIM), f32),
)


def _conv_params(p):
    return (p['w1'].T, p['b1'].reshape(1, -1), p['w2'].T,
            p['b2'].reshape(1, -1), p['w3'].T, p['b3'].reshape(1, -1))


def _em_params(p, xdim):
    w1t = p['w1'].T
    return (w1t[:xdim], w1t[xdim:2 * xdim], w1t[2 * xdim:],
            p['b1'].reshape(1, -1), p['gamma'].reshape(1, -1),
            p['beta'].reshape(1, -1), p['w2'].T, p['b2'].reshape(1, -1))


@jax.jit
def kernel(x, edge_index, edge_attr, params):
    row2d = edge_index[0].reshape(E // IDXW, IDXW)
    col2d = edge_index[1].reshape(E // IDXW, IDXW)
    zeros_n = jnp.zeros((N, DIM), f32)

    x0 = x.reshape(N, 1)
    table0 = jnp.concatenate([x0, jnp.zeros((N, DIM - 1), f32)], axis=1)
    x0_pad = jnp.concatenate([x, jnp.zeros((240,), f32)]).reshape(80, 128)
    stats = _stats0(x0_pad)
    (xs,) = _gather1(table0, row2d)
    ea = edge_attr
    x_cur = x0
    xr = xs
    ea_out = None

    for i in range(1, 7):
        ci = 1 if i == 1 else DIM
        co = 1 if i == 6 else DIM
        cp = params['conv%d' % i]
        conv = _conv_first if i == 1 else (_conv_last if i == 6 else _conv_mid)
        msg = conv(stats, ea, xr, *_conv_params(cp))
        partials = _scatter_add(msg, col2d, zeros_n)
        if ci == DIM and co == DIM:
            roott = cp['root']
        else:
            roott = cp['root'].reshape(1, -1)
        bias = cp['bias'].reshape(1, -1)
        if co == 1:
            bias = jnp.broadcast_to(bias, (1, DIM))
        if i == 1:
            x_cur, stats = _node_first(partials, x_cur, stats, roott, bias)
        elif i == 6:
            x_cur = _node_last(partials, x_cur, stats, roott, bias)
        else:
            x_cur, stats = _node_mid(partials, x_cur, stats, roott, bias)

        xr, xc = _gather2(x_cur, row2d, col2d)
        ep = params['em%d' % i]
        if i < 6:
            ea = _em_mid(ea, xr, xc, *_em_params(ep, DIM))
        else:
            ea_out = _em_last(ea, xr, xc, *_em_params(ep, 1))

    return x_cur[:, 0:1], ea_out[:, 0:1]


# speedup vs baseline: 4.2808x; 4.2808x over previous
import functools

import jax
import jax.numpy as jnp
from jax import lax
from jax.experimental import pallas as pl
from jax.experimental.pallas import tpu as pltpu
from jax.experimental.pallas import tpu_sc as plsc

N = 10000
E = 320000
DIM = 16
EPS = 1e-5

NC = 2
NS = 16
NW = NC * NS
EPW = E // NW
IDXW = 125
ROWS_PER_W = EPW // IDXW
CHUNK = 1000
DMAS_PER_CHUNK = CHUNK // IDXW
CHUNKS_PER_W = EPW // CHUNK

f32 = jnp.float32


def relu(v):
    return jnp.maximum(v, 0.0)


def _make_gather(n_idx):
    mesh = plsc.VectorSubcoreMesh(core_axis_name="c", subcore_axis_name="s",
                                  num_cores=NC, num_subcores=NS)
    out_type = [jax.ShapeDtypeStruct((E, DIM), f32)] * n_idx
    scratch = [pltpu.VMEM((ROWS_PER_W, IDXW), jnp.int32)] * n_idx
    scratch += [pltpu.VMEM((CHUNK, DIM), f32), pltpu.SemaphoreType.DMA]

    @functools.partial(pl.kernel, out_type=out_type, mesh=mesh,
                       scratch_types=scratch,
                       compiler_params=pltpu.CompilerParams(
                           use_tc_tiling_on_sc=False))
    def gather(*refs):
        table = refs[0]
        idx_hbm = refs[1:1 + n_idx]
        outs = refs[1 + n_idx:1 + 2 * n_idx]
        idx_v = refs[1 + 2 * n_idx:1 + 3 * n_idx]
        rows_v = refs[1 + 3 * n_idx]
        sem = refs[2 + 3 * n_idx]

        wid = lax.axis_index("s") * NC + lax.axis_index("c")
        irow0 = wid * ROWS_PER_W
        base = wid * EPW
        for t in range(n_idx):
            pltpu.sync_copy(idx_hbm[t].at[pl.ds(irow0, ROWS_PER_W), :],
                            idx_v[t])

        def body(j, carry):
            for t in range(n_idx):
                descs = []
                for k in range(DMAS_PER_CHUNK):
                    irow = j * DMAS_PER_CHUNK + k
                    d = pltpu.async_copy(
                        table.at[idx_v[t].at[irow]],
                        rows_v.at[pl.ds(k * IDXW, IDXW), :], sem)
                    descs.append(d)
                for d in descs:
                    d.wait()
                pltpu.sync_copy(rows_v,
                                outs[t].at[pl.ds(base + j * CHUNK, CHUNK), :])
            return carry

        lax.fori_loop(0, CHUNKS_PER_W, body, 0)

    return gather


_gather1 = _make_gather(1)
_gather2 = _make_gather(2)


def _scatter_add_kernel():
    mesh = plsc.VectorSubcoreMesh(core_axis_name="c", subcore_axis_name="s",
                                  num_cores=NC, num_subcores=NS)
    npc = N // NS

    @functools.partial(
        pl.kernel,
        out_type=jax.ShapeDtypeStruct((NC, N, DIM), f32),
        mesh=mesh,
        compiler_params=pltpu.CompilerParams(use_tc_tiling_on_sc=False),
        scratch_types=[
            pltpu.VMEM_SHARED((N, DIM), f32),
            pltpu.VMEM((ROWS_PER_W, IDXW), jnp.int32),
            pltpu.VMEM((CHUNK, DIM), f32),
        ])
    def scatter_add(msg_hbm, dst_hbm, zeros_hbm, out_hbm, acc_sh, idx_v,
                    msg_v):
        cid = lax.axis_index("c")
        sid = lax.axis_index("s")
        wid = sid * NC + cid
        pltpu.sync_copy(zeros_hbm.at[pl.ds(sid * npc, npc), :],
                        acc_sh.at[pl.ds(sid * npc, npc), :])
        plsc.subcore_barrier()

        pltpu.sync_copy(dst_hbm.at[pl.ds(wid * ROWS_PER_W, ROWS_PER_W), :],
                        idx_v)
        base = wid * EPW

        def body(j, carry):
            pltpu.sync_copy(msg_hbm.at[pl.ds(base + j * CHUNK, CHUNK), :],
                            msg_v)
            for k in range(DMAS_PER_CHUNK):
                irow = j * DMAS_PER_CHUNK + k
                pltpu.sync_copy(msg_v.at[pl.ds(k * IDXW, IDXW), :],
                                acc_sh.at[idx_v.at[irow]], add=True)
            return carry

        lax.fori_loop(0, CHUNKS_PER_W, body, 0)
        plsc.subcore_barrier()
        pltpu.sync_copy(acc_sh.at[pl.ds(sid * npc, npc), :],
                        out_hbm.at[cid, pl.ds(sid * npc, npc), :])

    return scatter_add


_scatter_add = _scatter_add_kernel()


BLK = 2560
NBLK = E // BLK


def _matT(w, xT):
    k = w.shape[1]
    acc = w[:, 0:1] * xT[0:1, :]
    for i in range(1, k):
        acc = acc + w[:, i:i + 1] * xT[i:i + 1, :]
    return acc


def _conv_body(ci, co, eaT_ref, xg_ref, w1, b1, w2, b2, w3, b3, msg_ref):
    eaT = eaT_ref[...]
    h1 = relu(_matT(w1[...], eaT) + b1[...])
    h2 = relu(_matT(w2[...], h1) + b2[...])
    w = relu(jnp.dot(w3[...], h2, preferred_element_type=f32,
                     precision=lax.Precision.HIGHEST) + b3[...])
    xnT = jnp.transpose(xg_ref[...])
    if ci == 1:
        msgT = xnT[0:1, :] * w
    elif co == DIM:
        parts = [xnT[i:i + 1, :] * w[i * DIM:(i + 1) * DIM, :]
                 for i in range(DIM)]
        while len(parts) > 1:
            parts = [parts[j] + parts[j + 1] for j in range(0, len(parts), 2)]
        msgT = parts[0]
    else:
        s = jnp.sum(xnT * w, axis=0, keepdims=True)
        msgT = jnp.concatenate([s, jnp.zeros((DIM - 1, s.shape[1]), f32)],
                               axis=0)
    msg_ref[...] = jnp.transpose(msgT)


def _make_conv(ci, co):
    wdim = ci * co
    body = functools.partial(_conv_body, ci, co)
    return pl.pallas_call(
        body,
        grid=(NBLK,),
        in_specs=[
            pl.BlockSpec((2, BLK), lambda e: (0, e)),
            pl.BlockSpec((BLK, DIM), lambda e: (e, 0)),
            pl.BlockSpec((4, 2), lambda e: (0, 0)),
            pl.BlockSpec((4, 1), lambda e: (0, 0)),
            pl.BlockSpec((DIM, 4), lambda e: (0, 0)),
            pl.BlockSpec((DIM, 1), lambda e: (0, 0)),
            pl.BlockSpec((wdim, DIM), lambda e: (0, 0)),
            pl.BlockSpec((wdim, 1), lambda e: (0, 0)),
        ],
        out_specs=pl.BlockSpec((BLK, DIM), lambda e: (e, 0)),
        out_shape=jax.ShapeDtypeStruct((E, DIM), f32),
    )


_conv_first = _make_conv(1, DIM)
_conv_mid = _make_conv(DIM, DIM)
_conv_last = _make_conv(DIM, 1)


def _em_body(folded, co, stats_ref, eaT_ref, xr_ref, xc_ref, w1a, w1b, w1c,
             b1, gamma, beta, w2, b2, out_ref):
    eaT = eaT_ref[...]
    if folded:
        st = jnp.transpose(stats_ref[...])
        mean = st[:, 0:1] * (1.0 / N)
        stdp = jnp.sqrt(st[:, 1:2] * (1.0 / N) - mean * mean + EPS)
        w1a_f = w1a[...] * jnp.transpose(stdp)
        w1b_f = w1b[...] * jnp.transpose(stdp)
        b1_f = (b1[...] + _matT(w1a[...], mean) + _matT(w1b[...], mean))
        dn = (((1,), (1,)), ((), ()))
        acc = (lax.dot_general(w1a_f, xr_ref[...], dn,
                               precision=lax.Precision.HIGHEST,
                               preferred_element_type=f32) +
               lax.dot_general(w1b_f, xc_ref[...], dn,
                               precision=lax.Precision.HIGHEST,
                               preferred_element_type=f32))
    else:
        b1_f = b1[...]
        xr0T = jnp.transpose(xr_ref[:, 0:1])
        xc0T = jnp.transpose(xc_ref[:, 0:1])
        acc = w1a[...] * xr0T + w1b[...] * xc0T
    acc = acc + _matT(w1c[...], eaT) + b1_f
    h = relu(acc)
    m = jnp.mean(h, axis=0, keepdims=True)
    v = jnp.mean((h - m) * (h - m), axis=0, keepdims=True)
    hn = (h - m) * (1.0 / jnp.sqrt(v + EPS)) * gamma[...] + beta[...]
    o = relu(jnp.dot(w2[...], hn, precision=lax.Precision.HIGHEST,
                     preferred_element_type=f32) + b2[...])
    if co == 2:
        out_ref[...] = o + eaT
    else:
        out_ref[...] = o


def _make_em(folded, co):
    body = functools.partial(_em_body, folded, co)
    xdim = DIM if folded else 1
    return pl.pallas_call(
        body,
        grid=(NBLK,),
        in_specs=[
            pl.BlockSpec((2, DIM), lambda e: (0, 0)),
            pl.BlockSpec((2, BLK), lambda e: (0, e)),
            pl.BlockSpec((BLK, DIM), lambda e: (e, 0)),
            pl.BlockSpec((BLK, DIM), lambda e: (e, 0)),
            pl.BlockSpec((DIM, xdim), lambda e: (0, 0)),
            pl.BlockSpec((DIM, xdim), lambda e: (0, 0)),
            pl.BlockSpec((DIM, 2), lambda e: (0, 0)),
            pl.BlockSpec((DIM, 1), lambda e: (0, 0)),
            pl.BlockSpec((DIM, 1), lambda e: (0, 0)),
            pl.BlockSpec((DIM, 1), lambda e: (0, 0)),
            pl.BlockSpec((co, DIM), lambda e: (0, 0)),
            pl.BlockSpec((co, 1), lambda e: (0, 0)),
        ],
        out_specs=pl.BlockSpec((co, BLK), lambda e: (0, e)),
        out_shape=jax.ShapeDtypeStruct((co, E), f32),
    )


_em_mid = _make_em(True, 2)
_em_last = _make_em(False, 1)


def _stats0_body(x_ref, out_ref):
    x = x_ref[...]
    s = jnp.sum(x)
    sq = jnp.sum(x * x)
    out_ref[...] = jnp.concatenate(
        [jnp.full((1, DIM), s, f32), jnp.full((1, DIM), sq, f32)], axis=0)


_stats0 = pl.pallas_call(
    _stats0_body,
    out_shape=jax.ShapeDtypeStruct((2, DIM), f32),
)


NODE_BLK = 1000
NODE_NBLK = N // NODE_BLK


def _mean_rstd(stats, d):
    s = stats[0:1, :d]
    sq = stats[1:2, :d]
    mean = s * (1.0 / N)
    var = sq * (1.0 / N) - mean * mean
    return mean, 1.0 / jnp.sqrt(var + EPS)


def _vpu_matmul(x, wt):
    k = wt.shape[0]
    acc = x[:, 0:1] * wt[0:1, :]
    for i in range(1, k):
        acc = acc + x[:, i:i + 1] * wt[i:i + 1, :]
    return acc


def _node_body(ci, p_ref, x_ref, stats_ref, roott, bias, x_out, st_out):
    agg = p_ref[0] + p_ref[1]
    mean, rstd = _mean_rstd(stats_ref[...], ci)
    xn = (x_ref[...] - mean) * rstd
    if ci == 1:
        r = xn * roott[...]
    else:
        r = _vpu_matmul(xn, roott[...])
    x_new = relu(agg + r + bias[...]) + x_ref[...]
    x_out[...] = x_new
    @pl.when(pl.program_id(0) == 0)
    def _():
        st_out[...] = jnp.zeros((2, DIM), f32)
    s = jnp.sum(x_new, axis=0, keepdims=True)
    sq = jnp.sum(x_new * x_new, axis=0, keepdims=True)
    st_out[...] += jnp.concatenate([s, sq], axis=0)


def _make_node(ci):
    xdim = 1 if ci == 1 else DIM
    return pl.pallas_call(
        functools.partial(_node_body, ci),
        grid=(NODE_NBLK,),
        in_specs=[
            pl.BlockSpec((NC, NODE_BLK, DIM), lambda n: (0, n, 0)),
            pl.BlockSpec((NODE_BLK, xdim), lambda n: (n, 0)),
            pl.BlockSpec((2, DIM), lambda n: (0, 0)),
            pl.BlockSpec((ci, DIM), lambda n: (0, 0)),
            pl.BlockSpec((1, DIM), lambda n: (0, 0)),
        ],
        out_specs=[
            pl.BlockSpec((NODE_BLK, DIM), lambda n: (n, 0)),
            pl.BlockSpec((2, DIM), lambda n: (0, 0)),
        ],
        out_shape=[
            jax.ShapeDtypeStruct((N, DIM), f32),
            jax.ShapeDtypeStruct((2, DIM), f32),
        ],
    )


_node_mid = _make_node(DIM)
_node_first = _make_node(1)


def _node6_body(p_ref, x_ref, stats_ref, roott, bias, x_out):
    agg = p_ref[0, :, 0:1] + p_ref[1, :, 0:1]
    mean, rstd = _mean_rstd(stats_ref[...], DIM)
    xn = (x_ref[...] - mean) * rstd
    r = jnp.sum(xn * roott[...], axis=1, keepdims=True)
    x6 = relu(agg + r + bias[:, 0:1])
    x_out[...] = jnp.concatenate(
        [x6, jnp.zeros((NODE_BLK, DIM - 1), f32)], axis=1)


_node_last = pl.pallas_call(
    _node6_body,
    grid=(NODE_NBLK,),
    in_specs=[
        pl.BlockSpec((NC, NODE_BLK, DIM), lambda n: (0, n, 0)),
        pl.BlockSpec((NODE_BLK, DIM), lambda n: (n, 0)),
        pl.BlockSpec((2, DIM), lambda n: (0, 0)),
        pl.BlockSpec((1, DIM), lambda n: (0, 0)),
        pl.BlockSpec((1, DIM), lambda n: (0, 0)),
    ],
    out_specs=pl.BlockSpec((NODE_BLK, DIM), lambda n: (n, 0)),
    out_shape=jax.ShapeDtypeStruct((N, DIM), f32),
)


def _norm_body(d, x_ref, stats_ref, xn_out):
    mean, rstd = _mean_rstd(stats_ref[...], d)
    xn = (x_ref[...] - mean) * rstd
    if d == 1:
        xn_out[...] = jnp.concatenate(
            [xn, jnp.zeros((NODE_BLK, DIM - 1), f32)], axis=1)
    else:
        xn_out[...] = xn


def _make_norm(d):
    return pl.pallas_call(
        functools.partial(_norm_body, d),
        grid=(NODE_NBLK,),
        in_specs=[
            pl.BlockSpec((NODE_BLK, d), lambda n: (n, 0)),
            pl.BlockSpec((2, DIM), lambda n: (0, 0)),
        ],
        out_specs=pl.BlockSpec((NODE_BLK, DIM), lambda n: (n, 0)),
        out_shape=jax.ShapeDtypeStruct((N, DIM), f32),
    )


_norm0 = _make_norm(1)
_norm = _make_norm(DIM)


def _conv_params(p):
    return (p['w1'], p['b1'].reshape(-1, 1), p['w2'], p['b2'].reshape(-1, 1),
            p['w3'], p['b3'].reshape(-1, 1))


def _em_params(p, xdim):
    w1 = p['w1']
    return (w1[:, :xdim], w1[:, xdim:2 * xdim], w1[:, 2 * xdim:],
            p['b1'].reshape(-1, 1), p['gamma'].reshape(-1, 1),
            p['beta'].reshape(-1, 1), p['w2'], p['b2'].reshape(-1, 1))


@jax.jit
def kernel(x, edge_index, edge_attr, params):
    row2d = edge_index[0].reshape(E // IDXW, IDXW)
    col2d = edge_index[1].reshape(E // IDXW, IDXW)
    zeros_n = jnp.zeros((N, DIM), f32)

    x0 = x.reshape(N, 1)
    x0_pad = jnp.concatenate([x, jnp.zeros((240,), f32)]).reshape(80, 128)
    stats = _stats0(x0_pad)
    xn_tab = _norm0(x0, stats)
    (xg,) = _gather1(xn_tab, row2d)
    eaT = jnp.transpose(edge_attr)
    x_cur = x0

    for i in range(1, 7):
        ci = 1 if i == 1 else DIM
        co = 1 if i == 6 else DIM
        cp = params['conv%d' % i]
        conv = _conv_first if i == 1 else (_conv_last if i == 6 else _conv_mid)
        msg = conv(eaT, xg, *_conv_params(cp))
        partials = _scatter_add(msg, col2d, zeros_n)
        if ci == DIM and co == DIM:
            roott = cp['root']
        else:
            roott = cp['root'].reshape(1, -1)
        bias = cp['bias'].reshape(1, -1)
        if co == 1:
            bias = jnp.broadcast_to(bias, (1, DIM))
        if i == 1:
            x_cur, stats = _node_first(partials, x_cur, stats, roott, bias)
        elif i == 6:
            x_cur = _node_last(partials, x_cur, stats, roott, bias)
        else:
            x_cur, stats = _node_mid(partials, x_cur, stats, roott, bias)

        ep = params['em%d' % i]
        if i < 6:
            xn_tab = _norm(x_cur, stats)
            xr, xc = _gather2(xn_tab, row2d, col2d)
            eaT = _em_mid(stats, eaT, xr, xc, *_em_params(ep, DIM))
            xg = xr
        else:
            xr, xc = _gather2(x_cur, row2d, col2d)
            ea_out = _em_last(stats, eaT, xr, xc, *_em_params(ep, 1))

    return x_cur[:, 0:1], jnp.transpose(ea_out)
